# Initial kernel scaffold; baseline (speedup 1.0000x reference)
import jax
import jax.numpy as jnp
from jax.experimental import pallas as pl


def kernel(features, W1, b1, W2, b2, det_emb, rec_emb):
    raise NotImplementedError("write your pallas kernel here")


# speedup vs baseline: 1.1190x; 1.1190x over previous
import functools

import jax
import jax.numpy as jnp
from jax.experimental import pallas as pl


def _body(fp_ref, w1_ref, b1_ref, w2_ref, b2_ref, out_ref, *, H, W, C, nq):
    HW = H * W
    fp = fp_ref[0]
    acc = jnp.zeros((HW, C), jnp.float32)
    for t in range(9):
        kh, kw = t // 3, t % 3
        a = fp[kh:kh + H, kw:kw + W, :].reshape(HW, C)
        acc = acc + jnp.dot(a, w1_ref[t], preferred_element_type=jnp.float32)
    x1 = jnp.maximum(acc + b1_ref[0][None, :], 0.0)
    logits = jnp.dot(x1, w2_ref[...], preferred_element_type=jnp.float32)
    pred = jax.nn.sigmoid(logits + b2_ref[0][None, :])

    s_col = pred[:, 4:5]
    s_row = jnp.transpose(pred)[4:5, :]

    rank = jnp.zeros((1, HW), jnp.int32)
    CHUNK = 256
    i_iota = jax.lax.broadcasted_iota(jnp.int32, (CHUNK, HW), 1)
    j_iota_base = jax.lax.broadcasted_iota(jnp.int32, (CHUNK, HW), 0)
    s_i = jnp.broadcast_to(s_row, (CHUNK, HW))
    for j0 in range(0, HW, CHUNK):
        s_j = jnp.broadcast_to(s_col[j0:j0 + CHUNK], (CHUNK, HW))
        j_iota = j_iota_base + j0
        m = (s_j > s_i) | ((s_j == s_i) & (j_iota < i_iota))
        rank = rank + jnp.sum(m.astype(jnp.int32), axis=0, keepdims=True)

    r_iota = jax.lax.broadcasted_iota(jnp.int32, (nq, HW), 0)
    onehot = (jnp.broadcast_to(rank, (nq, HW)) == r_iota).astype(jnp.float32)
    out_ref[0] = jnp.dot(onehot, pred, preferred_element_type=jnp.float32)


def kernel(features, W1, b1, W2, b2, det_emb, rec_emb):
    B, C, H, W = features.shape
    nq = det_emb.shape[0]

    f_nhwc = jnp.transpose(features, (0, 2, 3, 1))
    f_pad = jnp.pad(f_nhwc, ((0, 0), (1, 1), (1, 1), (0, 0)))
    w1 = jnp.transpose(W1, (2, 3, 1, 0)).reshape(9, C, C)
    w2 = jnp.pad(jnp.transpose(W2[:, :, 0, 0]), ((0, 0), (0, 123)))
    b1r = b1.reshape(1, C)
    b2r = jnp.pad(b2, (0, 123)).reshape(1, 128)

    out = pl.pallas_call(
        functools.partial(_body, H=H, W=W, C=C, nq=nq),
        grid=(B,),
        in_specs=[
            pl.BlockSpec((1, H + 2, W + 2, C), lambda b: (b, 0, 0, 0)),
            pl.BlockSpec((9, C, C), lambda b: (0, 0, 0)),
            pl.BlockSpec((1, C), lambda b: (0, 0)),
            pl.BlockSpec((C, 128), lambda b: (0, 0)),
            pl.BlockSpec((1, 128), lambda b: (0, 0)),
        ],
        out_specs=pl.BlockSpec((1, nq, 128), lambda b: (b, 0, 0)),
        out_shape=jax.ShapeDtypeStruct((B, nq, 128), jnp.float32),
    )(f_pad, w1, b1r, w2, b2r)

    coarse = out[:, :, :5]
    det_queries = jnp.broadcast_to(det_emb[None, :, :], (B, nq, C))
    rec_queries = jnp.broadcast_to(rec_emb[None, :, :], (B, nq, C))
    return (det_queries, rec_queries, coarse)
